# Initial kernel scaffold; baseline (speedup 1.0000x reference)
import jax
import jax.numpy as jnp
from jax.experimental import pallas as pl


def kernel(hyperedge, hyper_node, ve_affiliation, W, b):
    raise NotImplementedError("write your pallas kernel here")


# speedup vs baseline: 1.9434x; 1.9434x over previous
import functools

import jax
import jax.numpy as jnp
from jax import lax
from jax.experimental import pallas as pl
from jax.experimental.pallas import tpu as pltpu
from jax.experimental.pallas import tpu_sc as plsc

EDGE_IN = 128
NODE_OUT = 128
N_HE = 10000
N_INC = 320000

_NC = 2
_NS = 16
_NW = _NC * _NS
_BPW = N_INC // _NW
_C = 80
_NCHUNK = _BPW // _C


def _pe_body(he_ref, w1_ref, w2_ref, b_ref, pe1_ref, pe2_ref):
    he = he_ref[...]
    pe1_ref[...] = (
        jnp.dot(he, w1_ref[...], preferred_element_type=jnp.float32) + b_ref[...]
    )
    pe2_ref[...] = jnp.dot(he, w2_ref[...], preferred_element_type=jnp.float32)


def _fuse_body(hn_ref, g_ref, w3_ref, out_ref):
    acc = jnp.dot(hn_ref[...], w3_ref[...], preferred_element_type=jnp.float32)
    out_ref[...] = jnp.maximum(acc + g_ref[...], 0.0)


def _gather_sum_body(pe1_hbm, pe2_hbm, idx0_hbm, idx1_hbm, g_hbm,
                     idx0_v, idx1_v, rows1_v, rows2_v, sem1, sem2):
    wid = lax.axis_index("s") * _NC + lax.axis_index("c")
    base = wid * _BPW

    def chunk(k, carry):
        off = k * _C
        pltpu.sync_copy(idx0_hbm.at[pl.ds(base + off, _C)], idx0_v)
        pltpu.sync_copy(idx1_hbm.at[pl.ds(base + off, _C)], idx1_v)
        cp1 = pltpu.make_async_copy(pe1_hbm.at[idx0_v], rows1_v, sem1)
        cp2 = pltpu.make_async_copy(pe2_hbm.at[idx1_v], rows2_v, sem2)
        cp1.start()
        cp2.start()
        cp1.wait()
        cp2.wait()

        def addrow(r, c2):
            for j in range(EDGE_IN // 16):
                sl = pl.ds(j * 16, 16)
                rows1_v[r, sl] = rows1_v[r, sl] + rows2_v[r, sl]
            return c2

        lax.fori_loop(0, _C, addrow, 0, unroll=2)
        pltpu.sync_copy(rows1_v, g_hbm.at[pl.ds(base + off, _C)])
        return carry

    lax.fori_loop(0, _NCHUNK, chunk, 0)


def _gather_sum(pe1, pe2, idx0, idx1):
    mesh = plsc.VectorSubcoreMesh(
        core_axis_name="c", subcore_axis_name="s",
        num_cores=_NC, num_subcores=_NS)
    return pl.kernel(
        _gather_sum_body,
        out_type=jax.ShapeDtypeStruct((N_INC, NODE_OUT), jnp.float32),
        mesh=mesh,
        scratch_types=[
            pltpu.VMEM((_C,), jnp.int32),
            pltpu.VMEM((_C,), jnp.int32),
            pltpu.VMEM((_C, NODE_OUT), jnp.float32),
            pltpu.VMEM((_C, NODE_OUT), jnp.float32),
            pltpu.SemaphoreType.DMA,
            pltpu.SemaphoreType.DMA,
        ],
    )(pe1, pe2, idx0, idx1)


def kernel(hyperedge, hyper_node, ve_affiliation, W, b):
    idx0 = ve_affiliation[0].astype(jnp.int32)
    idx1 = ve_affiliation[1].astype(jnp.int32)
    w1 = W[:EDGE_IN]
    w2 = W[EDGE_IN:2 * EDGE_IN]
    w3 = W[2 * EDGE_IN:]
    b2 = b.reshape(1, NODE_OUT)

    pe1, pe2 = pl.pallas_call(
        _pe_body,
        out_shape=(
            jax.ShapeDtypeStruct((N_HE, NODE_OUT), jnp.float32),
            jax.ShapeDtypeStruct((N_HE, NODE_OUT), jnp.float32),
        ),
    )(hyperedge, w1, w2, b2)

    g = _gather_sum(pe1, pe2, idx0, idx1)

    blk = 2000
    out = pl.pallas_call(
        _fuse_body,
        grid=(N_INC // blk,),
        in_specs=[
            pl.BlockSpec((blk, EDGE_IN), lambda i: (i, 0)),
            pl.BlockSpec((blk, NODE_OUT), lambda i: (i, 0)),
            pl.BlockSpec((EDGE_IN, NODE_OUT), lambda i: (0, 0)),
        ],
        out_specs=pl.BlockSpec((blk, NODE_OUT), lambda i: (i, 0)),
        out_shape=jax.ShapeDtypeStruct((N_INC, NODE_OUT), jnp.float32),
    )(hyper_node, g, w3)
    return out


# speedup vs baseline: 2.8182x; 1.4501x over previous
import functools

import jax
import jax.numpy as jnp
from jax import lax
from jax.experimental import pallas as pl
from jax.experimental.pallas import tpu as pltpu
from jax.experimental.pallas import tpu_sc as plsc

EDGE_IN = 128
NODE_OUT = 128
N_HE = 10000
N_INC = 320000

_NC = 2
_NS = 16
_NW = _NC * _NS
_BPW = N_INC // _NW
_C = 80
_NCHUNK = _BPW // _C


def _pe_body(he_ref, w1_ref, w2_ref, b_ref, pe1_ref, pe2_ref):
    he = he_ref[...]
    pe1_ref[...] = (
        jnp.dot(he, w1_ref[...], preferred_element_type=jnp.float32) + b_ref[...]
    )
    pe2_ref[...] = jnp.dot(he, w2_ref[...], preferred_element_type=jnp.float32)


def _fuse_body(hn_ref, g_ref, w3_ref, out_ref):
    acc = jnp.dot(hn_ref[...], w3_ref[...], preferred_element_type=jnp.float32)
    out_ref[...] = jnp.maximum(acc + g_ref[...], 0.0)


def _gather_sum_body(pe1_hbm, pe2_hbm, idx0_hbm, idx1_hbm, g_hbm,
                     idx0_v, idx1_v, r1a, r1b, r2a, r2b, oa, ob,
                     gsema, gsemb, wsema, wsemb):
    wid = lax.axis_index("s") * _NC + lax.axis_index("c")
    base = wid * _BPW
    pltpu.sync_copy(idx0_hbm.at[pl.ds(base, _BPW)], idx0_v)
    pltpu.sync_copy(idx1_hbm.at[pl.ds(base, _BPW)], idx1_v)

    r1 = (r1a, r1b)
    r2 = (r2a, r2b)
    ob_ = (oa, ob)
    gsem = (gsema, gsemb)
    wsem = (wsema, wsemb)

    def fire_gather(k, p):
        pltpu.make_async_copy(
            pe1_hbm.at[idx0_v.at[pl.ds(k * _C, _C)]], r1[p], gsem[p]).start()
        pltpu.make_async_copy(
            pe2_hbm.at[idx1_v.at[pl.ds(k * _C, _C)]], r2[p], gsem[p]).start()

    def wait_gather(p):
        pltpu.make_async_copy(
            pe1_hbm.at[idx0_v.at[pl.ds(0, _C)]], r1[p], gsem[p]).wait()
        pltpu.make_async_copy(
            pe2_hbm.at[idx1_v.at[pl.ds(0, _C)]], r2[p], gsem[p]).wait()

    def compute(p):
        def addrow(r, c2):
            for j in range(NODE_OUT // 16):
                sl = pl.ds(j * 16, 16)
                ob_[p][r, sl] = r1[p][r, sl] + r2[p][r, sl]
            return c2
        lax.fori_loop(0, _C, addrow, 0, unroll=2)

    def fire_wb(k, p):
        pltpu.make_async_copy(
            ob_[p], g_hbm.at[pl.ds(base + k * _C, _C)], wsem[p]).start()

    def wait_wb(p):
        pltpu.make_async_copy(
            ob_[p], g_hbm.at[pl.ds(base, _C)], wsem[p]).wait()

    fire_gather(0, 0)

    def pair(m, carry):
        k0 = 2 * m
        fire_gather(k0 + 1, 1)
        wait_gather(0)

        @pl.when(m > 0)
        def _():
            wait_wb(0)
        compute(0)
        fire_wb(k0, 0)
        fire_gather(k0 + 2, 0)
        wait_gather(1)

        @pl.when(m > 0)
        def _():
            wait_wb(1)
        compute(1)
        fire_wb(k0 + 1, 1)
        return carry

    lax.fori_loop(0, (_NCHUNK - 1) // 2, pair, 0)
    wait_gather(0)
    wait_wb(0)
    compute(0)
    fire_wb(_NCHUNK - 1, 0)
    wait_wb(1)
    wait_wb(0)


def _gather_sum(pe1, pe2, idx0_2d, idx1_2d):
    mesh = plsc.VectorSubcoreMesh(
        core_axis_name="c", subcore_axis_name="s",
        num_cores=_NC, num_subcores=_NS)
    return pl.kernel(
        _gather_sum_body,
        out_type=jax.ShapeDtypeStruct((N_INC, NODE_OUT), jnp.float32),
        mesh=mesh,
        scratch_types=[
            pltpu.VMEM((_BPW,), jnp.int32),
            pltpu.VMEM((_BPW,), jnp.int32),
            pltpu.VMEM((_C, NODE_OUT), jnp.float32),
            pltpu.VMEM((_C, NODE_OUT), jnp.float32),
            pltpu.VMEM((_C, NODE_OUT), jnp.float32),
            pltpu.VMEM((_C, NODE_OUT), jnp.float32),
            pltpu.VMEM((_C, NODE_OUT), jnp.float32),
            pltpu.VMEM((_C, NODE_OUT), jnp.float32),
            pltpu.SemaphoreType.DMA,
            pltpu.SemaphoreType.DMA,
            pltpu.SemaphoreType.DMA,
            pltpu.SemaphoreType.DMA,
        ],
    )(pe1, pe2, idx0_2d, idx1_2d)


def kernel(hyperedge, hyper_node, ve_affiliation, W, b):
    idx0 = ve_affiliation[0].astype(jnp.int32)
    idx1 = ve_affiliation[1].astype(jnp.int32)
    w1 = W[:EDGE_IN]
    w2 = W[EDGE_IN:2 * EDGE_IN]
    w3 = W[2 * EDGE_IN:]
    b2 = b.reshape(1, NODE_OUT)

    pe1, pe2 = pl.pallas_call(
        _pe_body,
        out_shape=(
            jax.ShapeDtypeStruct((N_HE, NODE_OUT), jnp.float32),
            jax.ShapeDtypeStruct((N_HE, NODE_OUT), jnp.float32),
        ),
    )(hyperedge, w1, w2, b2)

    g = _gather_sum(pe1, pe2, idx0, idx1)

    blk = 2000
    out = pl.pallas_call(
        _fuse_body,
        grid=(N_INC // blk,),
        in_specs=[
            pl.BlockSpec((blk, EDGE_IN), lambda i: (i, 0)),
            pl.BlockSpec((blk, NODE_OUT), lambda i: (i, 0)),
            pl.BlockSpec((EDGE_IN, NODE_OUT), lambda i: (0, 0)),
        ],
        out_specs=pl.BlockSpec((blk, NODE_OUT), lambda i: (i, 0)),
        out_shape=jax.ShapeDtypeStruct((N_INC, NODE_OUT), jnp.float32),
    )(hyper_node, g, w3)
    return out


# speedup vs baseline: 3.3967x; 1.2053x over previous
import jax
import jax.numpy as jnp
from jax import lax
from jax.experimental import pallas as pl
from jax.experimental.pallas import tpu as pltpu
from jax.experimental.pallas import tpu_sc as plsc

EDGE_IN = 128
NODE_OUT = 128
N_HE = 10000
N_INC = 320000

_NC = 2
_NS = 16
_NW = _NC * _NS
_NSLICE = 5
_SLICE = N_INC // _NSLICE
_BPW = _SLICE // _NW
_C = 80
_NCHUNK = _BPW // _C
_BLK = 2000
_BLKS_PER_SLICE = _SLICE // _BLK


def _pe_body(he_ref, w1_ref, w2_ref, b_ref, pe1_ref, pe2_ref):
    he = he_ref[...]
    pe1_ref[...] = (
        jnp.dot(he, w1_ref[...], preferred_element_type=jnp.float32) + b_ref[...]
    )
    pe2_ref[...] = jnp.dot(he, w2_ref[...], preferred_element_type=jnp.float32)


def _fuse_body(hn_ref, g_ref, w3_ref, out_ref):
    acc = jnp.dot(hn_ref[...], w3_ref[...], preferred_element_type=jnp.float32)
    out_ref[...] = jnp.maximum(acc + g_ref[...], 0.0)


def _gather_sum_body(pe1_hbm, pe2_hbm, idx0_hbm, idx1_hbm, g_hbm,
                     idx0_v, idx1_v, r1a, r1b, r2a, r2b, oa, ob,
                     gsema, gsemb, wsema, wsemb):
    wid = lax.axis_index("s") * _NC + lax.axis_index("c")
    base = wid * _BPW
    pltpu.sync_copy(idx0_hbm.at[pl.ds(base, _BPW)], idx0_v)
    pltpu.sync_copy(idx1_hbm.at[pl.ds(base, _BPW)], idx1_v)

    r1 = (r1a, r1b)
    r2 = (r2a, r2b)
    ob_ = (oa, ob)
    gsem = (gsema, gsemb)
    wsem = (wsema, wsemb)

    def fire_gather(k, p):
        pltpu.make_async_copy(
            pe1_hbm.at[idx0_v.at[pl.ds(k * _C, _C)]], r1[p], gsem[p]).start()
        pltpu.make_async_copy(
            pe2_hbm.at[idx1_v.at[pl.ds(k * _C, _C)]], r2[p], gsem[p]).start()

    def wait_gather(p):
        pltpu.make_async_copy(
            pe1_hbm.at[idx0_v.at[pl.ds(0, _C)]], r1[p], gsem[p]).wait()
        pltpu.make_async_copy(
            pe2_hbm.at[idx1_v.at[pl.ds(0, _C)]], r2[p], gsem[p]).wait()

    def compute(p):
        def addrow(r, c2):
            for j in range(NODE_OUT // 16):
                sl = pl.ds(j * 16, 16)
                ob_[p][r, sl] = r1[p][r, sl] + r2[p][r, sl]
            return c2
        lax.fori_loop(0, _C, addrow, 0, unroll=2)

    def fire_wb(k, p):
        pltpu.make_async_copy(
            ob_[p], g_hbm.at[pl.ds(base + k * _C, _C)], wsem[p]).start()

    def wait_wb(p):
        pltpu.make_async_copy(
            ob_[p], g_hbm.at[pl.ds(base, _C)], wsem[p]).wait()

    fire_gather(0, 0)

    def pair(m, carry):
        k0 = 2 * m
        fire_gather(k0 + 1, 1)
        wait_gather(0)

        @pl.when(m > 0)
        def _():
            wait_wb(0)
        compute(0)
        fire_wb(k0, 0)
        fire_gather(k0 + 2, 0)
        wait_gather(1)

        @pl.when(m > 0)
        def _():
            wait_wb(1)
        compute(1)
        fire_wb(k0 + 1, 1)
        return carry

    lax.fori_loop(0, (_NCHUNK - 1) // 2, pair, 0)
    wait_gather(0)
    wait_wb(0)
    compute(0)
    fire_wb(_NCHUNK - 1, 0)
    wait_wb(1)
    wait_wb(0)


def _gather_sum(pe1, pe2, idx0_s, idx1_s):
    mesh = plsc.VectorSubcoreMesh(
        core_axis_name="c", subcore_axis_name="s",
        num_cores=_NC, num_subcores=_NS)
    return pl.kernel(
        _gather_sum_body,
        out_type=jax.ShapeDtypeStruct((_SLICE, NODE_OUT), jnp.float32),
        mesh=mesh,
        scratch_types=[
            pltpu.VMEM((_BPW,), jnp.int32),
            pltpu.VMEM((_BPW,), jnp.int32),
            pltpu.VMEM((_C, NODE_OUT), jnp.float32),
            pltpu.VMEM((_C, NODE_OUT), jnp.float32),
            pltpu.VMEM((_C, NODE_OUT), jnp.float32),
            pltpu.VMEM((_C, NODE_OUT), jnp.float32),
            pltpu.VMEM((_C, NODE_OUT), jnp.float32),
            pltpu.VMEM((_C, NODE_OUT), jnp.float32),
            pltpu.SemaphoreType.DMA,
            pltpu.SemaphoreType.DMA,
            pltpu.SemaphoreType.DMA,
            pltpu.SemaphoreType.DMA,
        ],
    )(pe1, pe2, idx0_s, idx1_s)


def kernel(hyperedge, hyper_node, ve_affiliation, W, b):
    idx0 = ve_affiliation[0].astype(jnp.int32)
    idx1 = ve_affiliation[1].astype(jnp.int32)
    w1 = W[:EDGE_IN]
    w2 = W[EDGE_IN:2 * EDGE_IN]
    w3 = W[2 * EDGE_IN:]
    b2 = b.reshape(1, NODE_OUT)

    pe1, pe2 = pl.pallas_call(
        _pe_body,
        out_shape=(
            jax.ShapeDtypeStruct((N_HE, NODE_OUT), jnp.float32),
            jax.ShapeDtypeStruct((N_HE, NODE_OUT), jnp.float32),
        ),
    )(hyperedge, w1, w2, b2)

    gs = [
        _gather_sum(pe1, pe2,
                    lax.slice_in_dim(idx0, s * _SLICE, (s + 1) * _SLICE),
                    lax.slice_in_dim(idx1, s * _SLICE, (s + 1) * _SLICE))
        for s in range(_NSLICE)
    ]

    out = None
    for s in range(_NSLICE):
        hn_spec = pl.BlockSpec((_BLK, EDGE_IN),
                               lambda i, s=s: (i + s * _BLKS_PER_SLICE, 0))
        out_spec = pl.BlockSpec((_BLK, NODE_OUT),
                                lambda i, s=s: (i + s * _BLKS_PER_SLICE, 0))
        in_specs = [
            hn_spec,
            pl.BlockSpec((_BLK, NODE_OUT), lambda i: (i, 0)),
            pl.BlockSpec((EDGE_IN, NODE_OUT), lambda i: (0, 0)),
        ]
        if s == 0:
            out = pl.pallas_call(
                _fuse_body,
                grid=(_BLKS_PER_SLICE,),
                in_specs=in_specs,
                out_specs=out_spec,
                out_shape=jax.ShapeDtypeStruct((N_INC, NODE_OUT), jnp.float32),
            )(hyper_node, gs[s], w3)
        else:
            def _fuse_acc_body(hn_ref, g_ref, w3_ref, prev_ref, out_ref):
                _fuse_body(hn_ref, g_ref, w3_ref, out_ref)

            out = pl.pallas_call(
                _fuse_acc_body,
                grid=(_BLKS_PER_SLICE,),
                in_specs=in_specs + [out_spec],
                out_specs=out_spec,
                out_shape=jax.ShapeDtypeStruct((N_INC, NODE_OUT), jnp.float32),
                input_output_aliases={3: 0},
            )(hyper_node, gs[s], w3, out)
    return out
